# baseline (speedup 1.0000x reference)
import functools

import jax
import jax.numpy as jnp
from jax import lax
from jax.experimental import pallas as pl
from jax.experimental.pallas import tpu as pltpu
from jax.experimental.pallas import tpu_sc as plsc

G = 1024
N = 64
F = 9
H = 64
NP1 = N + 1
L = 16

NC = 2
NS = 16
NW = NC * NS
GPW = G // NW
BG = 16
NB = GPW // BG


@functools.partial(
    pl.kernel,
    mesh=plsc.VectorSubcoreMesh(core_axis_name="c", subcore_axis_name="s"),
    out_type=jax.ShapeDtypeStruct((G, NP1, H), jnp.float32),
    scratch_types=[
        pltpu.VMEM((BG, N, F), jnp.int32),
        pltpu.VMEM((BG, F, N), jnp.int32),
        pltpu.VMEM((BG, N), jnp.int32),
        pltpu.VMEM((BG, NP1, H), jnp.float32),
        pltpu.SemaphoreType.DMA,
        pltpu.SemaphoreType.DMA,
    ],
    compiler_params=pltpu.CompilerParams(use_tc_tiling_on_sc=False,
                                         needs_layout_passes=False),
)
def _atom_feature_sc(af_hbm, dg_hbm, wa_hbm, wd_hbm, wv_hbm, out_hbm,
                     raw_v, atidx_v, didx_v, obuf_v, sem, sem2):
    wid = lax.axis_index("s") * NC + lax.axis_index("c")

    for k in range(BG):
        pltpu.async_copy(wv_hbm, obuf_v.at[k, pl.ds(0, 1)], sem2)
    for k in range(BG):
        pltpu.make_async_copy(wv_hbm, obuf_v.at[k, pl.ds(0, 1)], sem2).wait()

    lane = lax.iota(jnp.int32, L)

    def per_batch(b, carry):
        g0 = wid * GPW + b * BG
        pltpu.async_copy(af_hbm.at[pl.ds(g0, BG)], raw_v, sem2)
        pltpu.async_copy(dg_hbm.at[pl.ds(g0, BG)], didx_v, sem2)
        pltpu.make_async_copy(af_hbm.at[pl.ds(g0, BG)], raw_v, sem2).wait()
        pltpu.make_async_copy(dg_hbm.at[pl.ds(g0, BG)], didx_v, sem2).wait()

        def issue_deg(k, c):
            pltpu.async_copy(wd_hbm.at[didx_v.at[k]],
                             obuf_v.at[k, pl.ds(1, N)], sem)
            return c
        lax.fori_loop(0, BG, issue_deg, 0)

        def transpose_idx(k, c):
            kvec = jnp.full((L,), k, dtype=jnp.int32)
            for f in range(F):
                fvec = jnp.full((L,), f, dtype=jnp.int32)
                for cchunk in range(N // L):
                    nvec = lane + (cchunk * L)
                    atidx_v[k, f, pl.ds(cchunk * L, L)] = plsc.load_gather(
                        raw_v, [kvec, nvec, fvec])
            return c
        lax.fori_loop(0, BG, transpose_idx, 0)

        def drain_deg(k, c):
            pltpu.make_async_copy(wd_hbm.at[didx_v.at[k]],
                                  obuf_v.at[k, pl.ds(1, N)], sem).wait()
            return c
        lax.fori_loop(0, BG, drain_deg, 0)

        def issue_atom(k, c):
            for f in range(F):
                pltpu.async_copy(wa_hbm.at[atidx_v.at[k, f]],
                                 obuf_v.at[k, pl.ds(1, N)], sem, add=True)
            return c
        lax.fori_loop(0, BG, issue_atom, 0)

        def drain_atom(k, c):
            for f in range(F):
                pltpu.make_async_copy(wa_hbm.at[atidx_v.at[k, f]],
                                      obuf_v.at[k, pl.ds(1, N)], sem).wait()
            return c
        lax.fori_loop(0, BG, drain_atom, 0)

        pltpu.sync_copy(obuf_v, out_hbm.at[pl.ds(g0, BG)])
        return carry

    lax.fori_loop(0, NB, per_batch, 0)


def kernel(atom_feat, degree, W_atom, W_degree, W_vnode):
    return _atom_feature_sc(atom_feat, degree, W_atom, W_degree, W_vnode)


# speedup vs baseline: 1.2675x; 1.2675x over previous
import functools

import jax
import jax.numpy as jnp
from jax import lax
from jax.experimental import pallas as pl
from jax.experimental.pallas import tpu as pltpu
from jax.experimental.pallas import tpu_sc as plsc

G = 1024
N = 64
F = 9
H = 64
NP1 = N + 1
L = 16

NC = 2
NS = 16
NW = NC * NS
GPW = G // NW
BG = 16
NB = GPW // BG


@functools.partial(
    pl.kernel,
    mesh=plsc.VectorSubcoreMesh(core_axis_name="c", subcore_axis_name="s"),
    out_type=jax.ShapeDtypeStruct((G, NP1, H), jnp.float32),
    scratch_types=[
        pltpu.VMEM((F, N, BG), jnp.int32),
        pltpu.VMEM((N, BG), jnp.int32),
        pltpu.VMEM((BG, F, N), jnp.int32),
        pltpu.VMEM((BG, N), jnp.int32),
        pltpu.VMEM((BG, NP1, H), jnp.float32),
        pltpu.SemaphoreType.DMA,
        pltpu.SemaphoreType.DMA,
    ],
    compiler_params=pltpu.CompilerParams(use_tc_tiling_on_sc=False,
                                         needs_layout_passes=False),
)
def _atom_feature_sc(af_hbm, dg_hbm, wa_hbm, wd_hbm, wv_hbm, out_hbm,
                     araw_v, draw_v, atidx_v, didx_v, obuf_v, sem, sem2):
    wid = lax.axis_index("s") * NC + lax.axis_index("c")

    for k in range(BG):
        pltpu.async_copy(wv_hbm, obuf_v.at[k, pl.ds(0, 1)], sem2)
    for k in range(BG):
        pltpu.make_async_copy(wv_hbm, obuf_v.at[k, pl.ds(0, 1)], sem2).wait()

    lane = lax.iota(jnp.int32, L)

    def per_batch(b, carry):
        g0 = wid * GPW + b * BG
        pltpu.async_copy(af_hbm.at[:, :, pl.ds(g0, BG)], araw_v, sem2)
        pltpu.async_copy(dg_hbm.at[:, pl.ds(g0, BG)], draw_v, sem2)
        pltpu.make_async_copy(af_hbm.at[:, :, pl.ds(g0, BG)], araw_v, sem2).wait()
        pltpu.make_async_copy(dg_hbm.at[:, pl.ds(g0, BG)], draw_v, sem2).wait()

        def build_didx(k, c):
            kvec = jnp.full((L,), k, dtype=jnp.int32)
            for cchunk in range(N // L):
                nvec = lane + (cchunk * L)
                didx_v[k, pl.ds(cchunk * L, L)] = plsc.load_gather(
                    draw_v, [nvec, kvec])
            return c
        lax.fori_loop(0, BG, build_didx, 0)

        def issue_deg(k, c):
            pltpu.async_copy(wd_hbm.at[didx_v.at[k]],
                             obuf_v.at[k, pl.ds(1, N)], sem)
            return c
        lax.fori_loop(0, BG, issue_deg, 0)

        def build_atidx(k, c):
            kvec = jnp.full((L,), k, dtype=jnp.int32)
            for f in range(F):
                fvec = jnp.full((L,), f, dtype=jnp.int32)
                for cchunk in range(N // L):
                    nvec = lane + (cchunk * L)
                    atidx_v[k, f, pl.ds(cchunk * L, L)] = plsc.load_gather(
                        araw_v, [fvec, nvec, kvec])
            return c
        lax.fori_loop(0, BG, build_atidx, 0)

        def drain_deg(k, c):
            pltpu.make_async_copy(wd_hbm.at[didx_v.at[k]],
                                  obuf_v.at[k, pl.ds(1, N)], sem).wait()
            return c
        lax.fori_loop(0, BG, drain_deg, 0)

        def issue_atom(k, c):
            for f in range(F):
                pltpu.async_copy(wa_hbm.at[atidx_v.at[k, f]],
                                 obuf_v.at[k, pl.ds(1, N)], sem, add=True)
            return c
        lax.fori_loop(0, BG, issue_atom, 0)

        def drain_atom(k, c):
            for f in range(F):
                pltpu.make_async_copy(wa_hbm.at[atidx_v.at[k, f]],
                                      obuf_v.at[k, pl.ds(1, N)], sem).wait()
            return c
        lax.fori_loop(0, BG, drain_atom, 0)

        pltpu.sync_copy(obuf_v, out_hbm.at[pl.ds(g0, BG)])
        return carry

    lax.fori_loop(0, NB, per_batch, 0)


def kernel(atom_feat, degree, W_atom, W_degree, W_vnode):
    af_t = atom_feat.transpose(2, 1, 0)
    dg_t = degree.transpose(1, 0)
    return _atom_feature_sc(af_t, dg_t, W_atom, W_degree, W_vnode)
